# Initial kernel scaffold; baseline (speedup 1.0000x reference)
import jax
import jax.numpy as jnp
from jax.experimental import pallas as pl


def kernel(xyz1, xyz2, points1, points2, W1, b1, g1, be1, W2, b2, g2, be2):
    raise NotImplementedError("write your pallas kernel here")


# speedup vs baseline: 12.3885x; 12.3885x over previous
import functools

import jax
import jax.numpy as jnp
from jax.experimental import pallas as pl


_HI = jax.lax.Precision.HIGHEST


def _pass1_body(xyz1_ref, xyz2t_ref, p1_ref, p2_ref, w1t_ref, b1_ref,
                x1_ref, stats_ref, *, TN, S):
    b = pl.program_id(0)
    i = pl.program_id(1)

    @pl.when(jnp.logical_and(b == 0, i == 0))
    def _():
        stats_ref[...] = jnp.zeros_like(stats_ref)

    a = xyz1_ref[0]
    bt = xyz2t_ref[0]
    ax, ay, az = a[:, 0:1], a[:, 1:2], a[:, 2:3]
    bx, by, bz = bt[0:1, :], bt[1:2, :], bt[2:3, :]
    a_sq = ax * ax + ay * ay + az * az
    b_sq = bx * bx + by * by + bz * bz
    bf = lambda v: v.astype(jnp.bfloat16).astype(jnp.float32)
    cross = bf(ax) * bf(bx) + bf(ay) * bf(by) + bf(az) * bf(bz)
    d = a_sq + b_sq - 2.0 * cross

    iota = jax.lax.broadcasted_iota(jnp.int32, (TN, S), 1)
    inf = jnp.float32(jnp.inf)

    def extract_min(dcur):
        mval = jnp.min(dcur, axis=1, keepdims=True)
        midx = jnp.min(jnp.where(dcur == mval, iota, S), axis=1,
                       keepdims=True)
        dnext = jnp.where(iota == midx, inf, dcur)
        return mval, midx, dnext

    m1, i1, d = extract_min(d)
    m2, i2, d = extract_min(d)
    m3, i3, _ = extract_min(d)

    w1 = 1.0 / (m1 + 1e-8)
    w2 = 1.0 / (m2 + 1e-8)
    w3 = 1.0 / (m3 + 1e-8)
    wsum = w1 + w2 + w3
    w1, w2, w3 = w1 / wsum, w2 / wsum, w3 / wsum

    zero = jnp.float32(0.0)
    oh = (jnp.where(iota == i1, w1, zero)
          + jnp.where(iota == i2, w2, zero)
          + jnp.where(iota == i3, w3, zero))

    interp = jax.lax.dot(oh, p2_ref[0], precision=_HI,
                         preferred_element_type=jnp.float32)

    D1 = p1_ref.shape[2]
    x1 = (jax.lax.dot(p1_ref[0], w1t_ref[:D1, :], precision=_HI,
                      preferred_element_type=jnp.float32)
          + jax.lax.dot(interp, w1t_ref[D1:, :], precision=_HI,
                        preferred_element_type=jnp.float32)
          + b1_ref[...])
    x1_ref[0] = x1

    s = jnp.sum(x1, axis=0, keepdims=True)
    sq = jnp.sum(x1 * x1, axis=0, keepdims=True)
    stats_ref[0:1, :] = stats_ref[0:1, :] + s
    stats_ref[1:2, :] = stats_ref[1:2, :] + sq


def _pass2_body(x1_ref, sc_ref, sh_ref, w2t_ref, b2_ref, x2_ref, stats_ref):
    @pl.when(pl.program_id(0) == 0)
    def _():
        stats_ref[...] = jnp.zeros_like(stats_ref)

    h = jnp.maximum(x1_ref[...] * sc_ref[...] + sh_ref[...], 0.0)
    y = jax.lax.dot(h, w2t_ref[...], precision=_HI,
                    preferred_element_type=jnp.float32) + b2_ref[...]
    x2_ref[...] = y
    stats_ref[0:1, :] = stats_ref[0:1, :] + jnp.sum(y, axis=0, keepdims=True)
    stats_ref[1:2, :] = stats_ref[1:2, :] + jnp.sum(y * y, axis=0,
                                                    keepdims=True)


def _pass3_body(x2_ref, sc_ref, sh_ref, out_ref):
    out_ref[...] = jnp.maximum(x2_ref[...] * sc_ref[...] + sh_ref[...], 0.0)


def _affine(stats, gamma, beta, count):
    mean = stats[0] / count
    var = stats[1] / count - mean * mean
    scale = gamma / jnp.sqrt(var + 1e-5)
    shift = beta - mean * scale
    return scale[None, :], shift[None, :]


@jax.jit
def kernel(xyz1, xyz2, points1, points2, W1, b1, g1, be1, W2, b2, g2, be2):
    B, N, _ = xyz1.shape
    S = xyz2.shape[1]
    D1 = points1.shape[2]
    D2 = points2.shape[2]
    Cin = D1 + D2
    C = W1.shape[0]
    TN = 512

    xyz2t = jnp.transpose(xyz2, (0, 2, 1))
    w1t = jnp.transpose(W1)
    w2t = jnp.transpose(W2)
    b1r = b1[None, :]
    b2r = b2[None, :]

    x1, stats1 = pl.pallas_call(
        functools.partial(_pass1_body, TN=TN, S=S),
        grid=(B, N // TN),
        in_specs=[
            pl.BlockSpec((1, TN, 3), lambda b, i: (b, i, 0)),
            pl.BlockSpec((1, 3, S), lambda b, i: (b, 0, 0)),
            pl.BlockSpec((1, TN, D1), lambda b, i: (b, i, 0)),
            pl.BlockSpec((1, S, D2), lambda b, i: (b, 0, 0)),
            pl.BlockSpec((Cin, C), lambda b, i: (0, 0)),
            pl.BlockSpec((1, C), lambda b, i: (0, 0)),
        ],
        out_specs=[
            pl.BlockSpec((1, TN, C), lambda b, i: (b, i, 0)),
            pl.BlockSpec((8, C), lambda b, i: (0, 0)),
        ],
        out_shape=[
            jax.ShapeDtypeStruct((B, N, C), jnp.float32),
            jax.ShapeDtypeStruct((8, C), jnp.float32),
        ],
    )(xyz1, xyz2t, points1, points2, w1t, b1r)

    count = jnp.float32(B * N)
    sc1, sh1 = _affine(stats1, g1, be1, count)

    BN = B * N
    TN2 = 2048
    x1_2d = x1.reshape(BN, C)
    x2, stats2 = pl.pallas_call(
        _pass2_body,
        grid=(BN // TN2,),
        in_specs=[
            pl.BlockSpec((TN2, C), lambda i: (i, 0)),
            pl.BlockSpec((1, C), lambda i: (0, 0)),
            pl.BlockSpec((1, C), lambda i: (0, 0)),
            pl.BlockSpec((C, C), lambda i: (0, 0)),
            pl.BlockSpec((1, C), lambda i: (0, 0)),
        ],
        out_specs=[
            pl.BlockSpec((TN2, C), lambda i: (i, 0)),
            pl.BlockSpec((8, C), lambda i: (0, 0)),
        ],
        out_shape=[
            jax.ShapeDtypeStruct((BN, C), jnp.float32),
            jax.ShapeDtypeStruct((8, C), jnp.float32),
        ],
    )(x1_2d, sc1, sh1, w2t, b2r)

    sc2, sh2 = _affine(stats2, g2, be2, count)

    TN3 = 4096
    out = pl.pallas_call(
        _pass3_body,
        grid=(BN // TN3,),
        in_specs=[
            pl.BlockSpec((TN3, C), lambda i: (i, 0)),
            pl.BlockSpec((1, C), lambda i: (0, 0)),
            pl.BlockSpec((1, C), lambda i: (0, 0)),
        ],
        out_specs=pl.BlockSpec((TN3, C), lambda i: (i, 0)),
        out_shape=jax.ShapeDtypeStruct((BN, C), jnp.float32),
    )(x2, sc2, sh2)

    return out.reshape(B, N, C)


# speedup vs baseline: 18.1092x; 1.4618x over previous
import functools

import jax
import jax.numpy as jnp
from jax.experimental import pallas as pl


_HI = jax.lax.Precision.DEFAULT


def _pass1_body(xyz1_ref, xyz2t_ref, p1_ref, p2_ref, w1t_ref, b1_ref,
                x1_ref, stats_ref, *, TN, S):
    b = pl.program_id(0)
    i = pl.program_id(1)

    @pl.when(jnp.logical_and(b == 0, i == 0))
    def _():
        stats_ref[...] = jnp.zeros_like(stats_ref)

    a = xyz1_ref[0]
    bt = xyz2t_ref[0]
    ax, ay, az = a[:, 0:1], a[:, 1:2], a[:, 2:3]
    bx, by, bz = bt[0:1, :], bt[1:2, :], bt[2:3, :]
    a_sq = ax * ax + ay * ay + az * az
    b_sq = bx * bx + by * by + bz * bz
    bf = lambda v: v.astype(jnp.bfloat16).astype(jnp.float32)
    cross = bf(ax) * bf(bx) + bf(ay) * bf(by) + bf(az) * bf(bz)
    d = a_sq + b_sq - 2.0 * cross

    iota = jax.lax.broadcasted_iota(jnp.int32, (TN, S), 1)
    inf = jnp.float32(jnp.inf)

    def extract_min(dcur):
        mval = jnp.min(dcur, axis=1, keepdims=True)
        midx = jnp.min(jnp.where(dcur == mval, iota, S), axis=1,
                       keepdims=True)
        dnext = jnp.where(iota == midx, inf, dcur)
        return mval, midx, dnext

    m1, i1, d = extract_min(d)
    m2, i2, d = extract_min(d)
    m3, i3, _ = extract_min(d)

    w1 = 1.0 / (m1 + 1e-8)
    w2 = 1.0 / (m2 + 1e-8)
    w3 = 1.0 / (m3 + 1e-8)
    wsum = w1 + w2 + w3
    w1, w2, w3 = w1 / wsum, w2 / wsum, w3 / wsum

    zero = jnp.float32(0.0)
    oh = (jnp.where(iota == i1, w1, zero)
          + jnp.where(iota == i2, w2, zero)
          + jnp.where(iota == i3, w3, zero))

    p2 = p2_ref[0]
    oh_hi = oh.astype(jnp.bfloat16).astype(jnp.float32)
    oh_lo = oh - oh_hi
    p2_hi = p2.astype(jnp.bfloat16).astype(jnp.float32)
    p2_lo = p2 - p2_hi

    def _dot(a, b):
        return jax.lax.dot(a, b, precision=_HI,
                           preferred_element_type=jnp.float32)

    interp = _dot(oh_hi, p2_hi) + (_dot(oh_lo, p2_hi) + _dot(oh_hi, p2_lo))

    D1 = p1_ref.shape[2]
    x1 = (jax.lax.dot(p1_ref[0], w1t_ref[:D1, :], precision=_HI,
                      preferred_element_type=jnp.float32)
          + jax.lax.dot(interp, w1t_ref[D1:, :], precision=_HI,
                        preferred_element_type=jnp.float32)
          + b1_ref[...])
    x1_ref[0] = x1

    s = jnp.sum(x1, axis=0, keepdims=True)
    sq = jnp.sum(x1 * x1, axis=0, keepdims=True)
    stats_ref[0:1, :] = stats_ref[0:1, :] + s
    stats_ref[1:2, :] = stats_ref[1:2, :] + sq


def _pass2_body(x1_ref, sc_ref, sh_ref, w2t_ref, b2_ref, x2_ref, stats_ref):
    @pl.when(pl.program_id(0) == 0)
    def _():
        stats_ref[...] = jnp.zeros_like(stats_ref)

    h = jnp.maximum(x1_ref[...] * sc_ref[...] + sh_ref[...], 0.0)
    y = jax.lax.dot(h, w2t_ref[...], precision=_HI,
                    preferred_element_type=jnp.float32) + b2_ref[...]
    x2_ref[...] = y
    stats_ref[0:1, :] = stats_ref[0:1, :] + jnp.sum(y, axis=0, keepdims=True)
    stats_ref[1:2, :] = stats_ref[1:2, :] + jnp.sum(y * y, axis=0,
                                                    keepdims=True)


def _pass3_body(x2_ref, sc_ref, sh_ref, out_ref):
    out_ref[...] = jnp.maximum(x2_ref[...] * sc_ref[...] + sh_ref[...], 0.0)


def _affine(stats, gamma, beta, count):
    mean = stats[0] / count
    var = stats[1] / count - mean * mean
    scale = gamma / jnp.sqrt(var + 1e-5)
    shift = beta - mean * scale
    return scale[None, :], shift[None, :]


@jax.jit
def kernel(xyz1, xyz2, points1, points2, W1, b1, g1, be1, W2, b2, g2, be2):
    B, N, _ = xyz1.shape
    S = xyz2.shape[1]
    D1 = points1.shape[2]
    D2 = points2.shape[2]
    Cin = D1 + D2
    C = W1.shape[0]
    TN = 512

    xyz2t = jnp.transpose(xyz2, (0, 2, 1))
    w1t = jnp.transpose(W1)
    w2t = jnp.transpose(W2)
    b1r = b1[None, :]
    b2r = b2[None, :]

    x1, stats1 = pl.pallas_call(
        functools.partial(_pass1_body, TN=TN, S=S),
        grid=(B, N // TN),
        in_specs=[
            pl.BlockSpec((1, TN, 3), lambda b, i: (b, i, 0)),
            pl.BlockSpec((1, 3, S), lambda b, i: (b, 0, 0)),
            pl.BlockSpec((1, TN, D1), lambda b, i: (b, i, 0)),
            pl.BlockSpec((1, S, D2), lambda b, i: (b, 0, 0)),
            pl.BlockSpec((Cin, C), lambda b, i: (0, 0)),
            pl.BlockSpec((1, C), lambda b, i: (0, 0)),
        ],
        out_specs=[
            pl.BlockSpec((1, TN, C), lambda b, i: (b, i, 0)),
            pl.BlockSpec((8, C), lambda b, i: (0, 0)),
        ],
        out_shape=[
            jax.ShapeDtypeStruct((B, N, C), jnp.float32),
            jax.ShapeDtypeStruct((8, C), jnp.float32),
        ],
    )(xyz1, xyz2t, points1, points2, w1t, b1r)

    count = jnp.float32(B * N)
    sc1, sh1 = _affine(stats1, g1, be1, count)

    BN = B * N
    TN2 = 2048
    x1_2d = x1.reshape(BN, C)
    x2, stats2 = pl.pallas_call(
        _pass2_body,
        grid=(BN // TN2,),
        in_specs=[
            pl.BlockSpec((TN2, C), lambda i: (i, 0)),
            pl.BlockSpec((1, C), lambda i: (0, 0)),
            pl.BlockSpec((1, C), lambda i: (0, 0)),
            pl.BlockSpec((C, C), lambda i: (0, 0)),
            pl.BlockSpec((1, C), lambda i: (0, 0)),
        ],
        out_specs=[
            pl.BlockSpec((TN2, C), lambda i: (i, 0)),
            pl.BlockSpec((8, C), lambda i: (0, 0)),
        ],
        out_shape=[
            jax.ShapeDtypeStruct((BN, C), jnp.float32),
            jax.ShapeDtypeStruct((8, C), jnp.float32),
        ],
    )(x1_2d, sc1, sh1, w2t, b2r)

    sc2, sh2 = _affine(stats2, g2, be2, count)

    TN3 = 4096
    out = pl.pallas_call(
        _pass3_body,
        grid=(BN // TN3,),
        in_specs=[
            pl.BlockSpec((TN3, C), lambda i: (i, 0)),
            pl.BlockSpec((1, C), lambda i: (0, 0)),
            pl.BlockSpec((1, C), lambda i: (0, 0)),
        ],
        out_specs=pl.BlockSpec((TN3, C), lambda i: (i, 0)),
        out_shape=jax.ShapeDtypeStruct((BN, C), jnp.float32),
    )(x2, sc2, sh2)

    return out.reshape(B, N, C)


# speedup vs baseline: 27.4215x; 1.5142x over previous
import functools

import jax
import jax.numpy as jnp
from jax.experimental import pallas as pl


_HI = jax.lax.Precision.DEFAULT


def _pass1_body(xyz1_ref, xyz2t_ref, p1_ref, p2_ref, w1t_ref, b1_ref,
                x1_ref, stats_ref, *, TN, S):
    b = pl.program_id(0)
    i = pl.program_id(1)

    @pl.when(jnp.logical_and(b == 0, i == 0))
    def _():
        stats_ref[...] = jnp.zeros_like(stats_ref)

    a = xyz1_ref[0]
    bt = xyz2t_ref[0]
    ax, ay, az = a[:, 0:1], a[:, 1:2], a[:, 2:3]
    bx, by, bz = bt[0:1, :], bt[1:2, :], bt[2:3, :]
    a_sq = ax * ax + ay * ay + az * az
    b_sq = bx * bx + by * by + bz * bz
    cross = jax.lax.dot(a, bt, precision=_HI,
                        preferred_element_type=jnp.float32)
    d = a_sq + b_sq - 2.0 * cross

    inf = jnp.float32(jnp.inf)
    m1 = jnp.min(d, axis=1, keepdims=True)
    d2 = jnp.where(d <= m1, inf, d)
    m2 = jnp.min(d2, axis=1, keepdims=True)
    d3 = jnp.where(d2 <= m2, inf, d2)
    m3 = jnp.min(d3, axis=1, keepdims=True)

    w1 = 1.0 / (m1 + 1e-8)
    w2 = 1.0 / (m2 + 1e-8)
    w3 = 1.0 / (m3 + 1e-8)
    wsum = w1 + w2 + w3
    w1, w2, w3 = w1 / wsum, w2 / wsum, w3 / wsum

    zero = jnp.float32(0.0)
    oh = jnp.where(d == m1, w1,
                   jnp.where(d == m2, w2,
                             jnp.where(d == m3, w3, zero)))

    p2 = p2_ref[0]
    oh_hi = oh.astype(jnp.bfloat16).astype(jnp.float32)
    oh_lo = oh - oh_hi
    p2_hi = p2.astype(jnp.bfloat16).astype(jnp.float32)
    p2_lo = p2 - p2_hi

    def _dot(a, b):
        return jax.lax.dot(a, b, precision=_HI,
                           preferred_element_type=jnp.float32)

    interp = _dot(oh_hi, p2_hi) + (_dot(oh_lo, p2_hi) + _dot(oh_hi, p2_lo))

    D1 = p1_ref.shape[2]
    x1 = (jax.lax.dot(p1_ref[0], w1t_ref[:D1, :], precision=_HI,
                      preferred_element_type=jnp.float32)
          + jax.lax.dot(interp, w1t_ref[D1:, :], precision=_HI,
                        preferred_element_type=jnp.float32)
          + b1_ref[...])
    x1_ref[0] = x1

    x1sq = x1 * x1
    s = x1[0:8, :]
    sq = x1sq[0:8, :]
    for r in range(8, TN, 8):
        s = s + x1[r:r + 8, :]
        sq = sq + x1sq[r:r + 8, :]
    stats_ref[0:8, :] = stats_ref[0:8, :] + s
    stats_ref[8:16, :] = stats_ref[8:16, :] + sq


def _pass2_body(x1_ref, sc_ref, sh_ref, w2t_ref, b2_ref, x2_ref, stats_ref):
    @pl.when(pl.program_id(0) == 0)
    def _():
        stats_ref[...] = jnp.zeros_like(stats_ref)

    h = jnp.maximum(x1_ref[...] * sc_ref[...] + sh_ref[...], 0.0)
    y = jax.lax.dot(h, w2t_ref[...], precision=_HI,
                    preferred_element_type=jnp.float32) + b2_ref[...]
    x2_ref[...] = y
    ysq = y * y
    TN2 = y.shape[0]
    s = y[0:8, :]
    sq = ysq[0:8, :]
    for r in range(8, TN2, 8):
        s = s + y[r:r + 8, :]
        sq = sq + ysq[r:r + 8, :]
    stats_ref[0:8, :] = stats_ref[0:8, :] + s
    stats_ref[8:16, :] = stats_ref[8:16, :] + sq


def _pass3_body(x2_ref, sc_ref, sh_ref, out_ref):
    out_ref[...] = jnp.maximum(x2_ref[...] * sc_ref[...] + sh_ref[...], 0.0)


def _affine(stats, gamma, beta, count):
    mean = jnp.sum(stats[0:8], axis=0) / count
    var = jnp.sum(stats[8:16], axis=0) / count - mean * mean
    scale = gamma / jnp.sqrt(var + 1e-5)
    shift = beta - mean * scale
    return scale[None, :], shift[None, :]


@jax.jit
def kernel(xyz1, xyz2, points1, points2, W1, b1, g1, be1, W2, b2, g2, be2):
    B, N, _ = xyz1.shape
    S = xyz2.shape[1]
    D1 = points1.shape[2]
    D2 = points2.shape[2]
    Cin = D1 + D2
    C = W1.shape[0]
    TN = 512

    xyz2t = jnp.transpose(xyz2, (0, 2, 1))
    w1t = jnp.transpose(W1)
    w2t = jnp.transpose(W2)
    b1r = b1[None, :]
    b2r = b2[None, :]

    x1, stats1 = pl.pallas_call(
        functools.partial(_pass1_body, TN=TN, S=S),
        grid=(B, N // TN),
        in_specs=[
            pl.BlockSpec((1, TN, 3), lambda b, i: (b, i, 0)),
            pl.BlockSpec((1, 3, S), lambda b, i: (b, 0, 0)),
            pl.BlockSpec((1, TN, D1), lambda b, i: (b, i, 0)),
            pl.BlockSpec((1, S, D2), lambda b, i: (b, 0, 0)),
            pl.BlockSpec((Cin, C), lambda b, i: (0, 0)),
            pl.BlockSpec((1, C), lambda b, i: (0, 0)),
        ],
        out_specs=[
            pl.BlockSpec((1, TN, C), lambda b, i: (b, i, 0)),
            pl.BlockSpec((16, C), lambda b, i: (0, 0)),
        ],
        out_shape=[
            jax.ShapeDtypeStruct((B, N, C), jnp.float32),
            jax.ShapeDtypeStruct((16, C), jnp.float32),
        ],
    )(xyz1, xyz2t, points1, points2, w1t, b1r)

    count = jnp.float32(B * N)
    sc1, sh1 = _affine(stats1, g1, be1, count)

    BN = B * N
    TN2 = 2048
    x1_2d = x1.reshape(BN, C)
    x2, stats2 = pl.pallas_call(
        _pass2_body,
        grid=(BN // TN2,),
        in_specs=[
            pl.BlockSpec((TN2, C), lambda i: (i, 0)),
            pl.BlockSpec((1, C), lambda i: (0, 0)),
            pl.BlockSpec((1, C), lambda i: (0, 0)),
            pl.BlockSpec((C, C), lambda i: (0, 0)),
            pl.BlockSpec((1, C), lambda i: (0, 0)),
        ],
        out_specs=[
            pl.BlockSpec((TN2, C), lambda i: (i, 0)),
            pl.BlockSpec((16, C), lambda i: (0, 0)),
        ],
        out_shape=[
            jax.ShapeDtypeStruct((BN, C), jnp.float32),
            jax.ShapeDtypeStruct((16, C), jnp.float32),
        ],
    )(x1_2d, sc1, sh1, w2t, b2r)

    sc2, sh2 = _affine(stats2, g2, be2, count)

    TN3 = 4096
    out = pl.pallas_call(
        _pass3_body,
        grid=(BN // TN3,),
        in_specs=[
            pl.BlockSpec((TN3, C), lambda i: (i, 0)),
            pl.BlockSpec((1, C), lambda i: (0, 0)),
            pl.BlockSpec((1, C), lambda i: (0, 0)),
        ],
        out_specs=pl.BlockSpec((TN3, C), lambda i: (i, 0)),
        out_shape=jax.ShapeDtypeStruct((BN, C), jnp.float32),
    )(x2, sc2, sh2)

    return out.reshape(B, N, C)
